# baseline (speedup 1.0000x reference)
import jax
import jax.numpy as jnp
from jax import lax
from jax.experimental import pallas as pl
from jax.experimental.pallas import tpu as pltpu
from jax.experimental.pallas import tpu_sc as plsc

N = 10000
E = 320000
D = 128
DH = D // 2
NC = 2
NS = 16
NP = 10240
NPER = NP // NS
RPW = 160
ROWS_PAD = NS * RPW
NBUF = 4

_MESH = plsc.VectorSubcoreMesh(
    core_axis_name="c", subcore_axis_name="s", num_cores=NC, num_subcores=NS
)


def _make_sc_agg(with_cnt: bool):
  out_type = [jax.ShapeDtypeStruct((NC, NP, DH), jnp.float32)]
  if with_cnt:
    out_type.append(jax.ShapeDtypeStruct((NC, NP), jnp.float32))

  scratch = [
      pltpu.VMEM((RPW, 128), jnp.int32),
      pltpu.VMEM((RPW, 128), jnp.int32),
      pltpu.VMEM((NBUF, 128, DH), jnp.float32),
      pltpu.VMEM((128,), jnp.float32),
      pltpu.VMEM_SHARED((NP, DH), jnp.float32),
      pltpu.VMEM_SHARED((NP,), jnp.float32),
  ] + [pltpu.SemaphoreType.DMA] * NBUF

  def body(x0_hbm, x1_hbm, src_hbm, dst_hbm, zeros_hbm, zeros_n_hbm,
           ones_hbm, *rest):
    if with_cnt:
      agg_out, cnt_out = rest[0], rest[1]
      rest = rest[2:]
    else:
      agg_out, cnt_out = rest[0], None
      rest = rest[1:]
    sidx, didx, rows, ones_v, agg_sh, cnt_sh = rest[:6]
    sems = rest[6:6 + NBUF]

    cid = lax.axis_index("c")
    sid = lax.axis_index("s")
    lo = sid * RPW

    pltpu.sync_copy(src_hbm.at[pl.ds(lo, RPW)], sidx)
    pltpu.sync_copy(dst_hbm.at[pl.ds(lo, RPW)], didx)

    pltpu.sync_copy(zeros_hbm.at[pl.ds(sid * NPER, NPER)],
                    agg_sh.at[pl.ds(sid * NPER, NPER)])
    if with_cnt:
      pltpu.sync_copy(zeros_n_hbm.at[pl.ds(sid * NPER, NPER)],
                      cnt_sh.at[pl.ds(sid * NPER, NPER)])
      pltpu.sync_copy(ones_hbm, ones_v)
    plsc.subcore_barrier()

    def gather(r, b):
      @pl.when(cid == 0)
      def _():
        pltpu.async_copy(x0_hbm.at[sidx.at[r]], rows.at[b], sems[b])

      @pl.when(cid == 1)
      def _():
        pltpu.async_copy(x1_hbm.at[sidx.at[r]], rows.at[b], sems[b])

    for b in range(NBUF):
      gather(b, b)

    def outer(i, carry):
      g = i * NBUF
      for b in range(NBUF):
        r = g + b
        pltpu.make_async_copy(x0_hbm.at[sidx.at[r]], rows.at[b],
                              sems[b]).wait()
        pltpu.sync_copy(rows.at[b], agg_sh.at[didx.at[r]], add=True)
        if with_cnt:
          pltpu.sync_copy(ones_v, cnt_sh.at[didx.at[r]], add=True)
        nxt = r + NBUF

        @pl.when(nxt < RPW)
        def _():
          gather(nxt, b)
      return carry

    lax.fori_loop(0, RPW // NBUF, outer, 0)
    plsc.subcore_barrier()

    pltpu.sync_copy(agg_sh.at[pl.ds(sid * NPER, NPER)],
                    agg_out.at[cid, pl.ds(sid * NPER, NPER)])
    if with_cnt:
      pltpu.sync_copy(cnt_sh.at[pl.ds(sid * NPER, NPER)],
                      cnt_out.at[cid, pl.ds(sid * NPER, NPER)])

  return pl.kernel(body, out_type=tuple(out_type), mesh=_MESH,
                   scratch_types=scratch,
                   compiler_params=pltpu.CompilerParams(
                       use_tc_tiling_on_sc=False))


_sc_agg_cnt = _make_sc_agg(with_cnt=True)
_sc_agg = _make_sc_agg(with_cnt=False)

BN = 1000


def _tc_layer1_body(al, ah, cf, x, wl, wr, b, olo, ohi):
  c = jnp.maximum(cf[...], 1.0)
  m = jnp.concatenate([al[...], ah[...]], axis=1) / c
  acc = jnp.dot(m, wl[...], preferred_element_type=jnp.float32)
  acc += jnp.dot(x[...], wr[...], preferred_element_type=jnp.float32)
  h = jnp.maximum(acc + b[...], 0.0)
  olo[...] = h[:, :DH]
  ohi[...] = h[:, DH:]


def _tc_layer2_body(al, ah, cf, xl, xh, wl, wr, b, lw, lb, o):
  c = jnp.maximum(cf[...], 1.0)
  m = jnp.concatenate([al[...], ah[...]], axis=1) / c
  x = jnp.concatenate([xl[...], xh[...]], axis=1)
  acc = jnp.dot(m, wl[...], preferred_element_type=jnp.float32)
  acc += jnp.dot(x[...], wr[...], preferred_element_type=jnp.float32)
  h = jnp.maximum(acc + b[...], 0.0)
  o[...] = jnp.dot(h, lw[...], preferred_element_type=jnp.float32) + lb[...]


_ROW_SPEC = pl.BlockSpec((BN, D), lambda i: (i, 0))
_HALF_SPEC = pl.BlockSpec((BN, DH), lambda i: (i, 0))
_CNT_SPEC = pl.BlockSpec((BN, 1), lambda i: (i, 0))
_W_SPEC = pl.BlockSpec((D, D), lambda i: (0, 0))
_B_SPEC = pl.BlockSpec((1, D), lambda i: (0, 0))

_tc_layer1 = pl.pallas_call(
    _tc_layer1_body,
    grid=(N // BN,),
    in_specs=[_HALF_SPEC, _HALF_SPEC, _CNT_SPEC, _ROW_SPEC,
              _W_SPEC, _W_SPEC, _B_SPEC],
    out_specs=(_HALF_SPEC, _HALF_SPEC),
    out_shape=(jax.ShapeDtypeStruct((N, DH), jnp.float32),
               jax.ShapeDtypeStruct((N, DH), jnp.float32)),
)

_tc_layer2 = pl.pallas_call(
    _tc_layer2_body,
    grid=(N // BN,),
    in_specs=[_HALF_SPEC, _HALF_SPEC, _CNT_SPEC, _HALF_SPEC, _HALF_SPEC,
              _W_SPEC, _W_SPEC, _B_SPEC,
              pl.BlockSpec((D, 1), lambda i: (0, 0)),
              pl.BlockSpec((1, 1), lambda i: (0, 0))],
    out_specs=pl.BlockSpec((BN, 1), lambda i: (i, 0)),
    out_shape=jax.ShapeDtypeStruct((N, 1), jnp.float32),
)


def kernel(x, edge_index, W1l, W1r, b1, W2l, W2r, b2, lin_W, lin_b):
  pad = ROWS_PAD * 128 - E
  src_r = jnp.concatenate(
      [edge_index[0], jnp.zeros((pad,), jnp.int32)]).reshape(ROWS_PAD, 128)
  dst_r = jnp.concatenate(
      [edge_index[1], jnp.full((pad,), NP - 1, jnp.int32)]).reshape(
          ROWS_PAD, 128)
  zeros = jnp.zeros((NP, DH), jnp.float32)
  zeros_n = jnp.zeros((NP,), jnp.float32)
  ones = jnp.ones((128,), jnp.float32)

  x0 = x[:, :DH]
  x1 = x[:, DH:]
  agg1, cnt = _sc_agg_cnt(x0, x1, src_r, dst_r, zeros, zeros_n, ones)
  cf = cnt[0, :N].reshape(N, 1)
  h1lo, h1hi = _tc_layer1(agg1[0, :N], agg1[1, :N], cf, x, W1l, W1r,
                          b1.reshape(1, D))

  (agg2,) = _sc_agg(h1lo, h1hi, src_r, dst_r, zeros, zeros_n, ones)
  out = _tc_layer2(agg2[0, :N], agg2[1, :N], cf, h1lo, h1hi, W2l, W2r,
                   b2.reshape(1, D), lin_W, lin_b.reshape(1, 1))
  return out
